# Initial kernel scaffold; baseline (speedup 1.0000x reference)
import jax
import jax.numpy as jnp
from jax.experimental import pallas as pl


def kernel(H, X, X_edges, P_w, P_b, a_w, a_b):
    raise NotImplementedError("write your pallas kernel here")


# speedup vs baseline: 6.8931x; 6.8931x over previous
import jax
import jax.numpy as jnp
from jax import lax
from jax.experimental import pallas as pl
from jax.experimental.pallas import tpu as pltpu
from jax.experimental.pallas import tpu_sc as plsc

N = 10000
E = 10000
NNZ = 320000
D = 128
D2 = 64

NP = 10240
C = 128
NW = 32
CPW = 80
NNZ_P = NW * CPW * C
NCHUNK = NNZ_P // C
NS = 16
RPT = NP // NS

f32 = jnp.float32
i32 = jnp.int32

_SC_PARAMS = pltpu.CompilerParams(use_tc_tiling_on_sc=False,
                                  needs_layout_passes=False)


def _mesh():
    return plsc.VectorSubcoreMesh(core_axis_name="c", subcore_axis_name="s")


def _dense_body(x_ref, xe_ref, pwt_ref, pb_ref, w1_ref, w2_ref, ab_ref,
                z_ref, ze_ref, pr_ref, pc_ref, m_ref):
    z = jnp.dot(x_ref[...], pwt_ref[...], preferred_element_type=f32) + pb_ref[...]
    ze = jnp.dot(xe_ref[...], pwt_ref[...], preferred_element_type=f32) + pb_ref[...]
    z_ref[...] = z
    ze_ref[...] = ze
    pr = jnp.dot(z, w1_ref[...], preferred_element_type=f32)
    pc = jnp.dot(ze, w2_ref[...], preferred_element_type=f32) + ab_ref[...]
    pr_ref[...] = pr
    pc_ref[...] = pc
    m = jnp.max(pr) + jnp.max(pc)
    m = jnp.where(m >= 0, m, 0.2 * m)
    m_ref[...] = jnp.full((1, 1), m, f32)


def _dense(X, X_edges, P_w, P_b, a_w, a_b):
    return pl.pallas_call(
        _dense_body,
        out_shape=[
            jax.ShapeDtypeStruct((N, D), f32),
            jax.ShapeDtypeStruct((E, D), f32),
            jax.ShapeDtypeStruct((N, 1), f32),
            jax.ShapeDtypeStruct((E, 1), f32),
            jax.ShapeDtypeStruct((1, 1), f32),
        ],
    )(X, X_edges, P_w.T, P_b.reshape(1, D), a_w[0, :D].reshape(D, 1),
      a_w[0, D:].reshape(D, 1), a_b.reshape(1, 1))


def _phase2_body(row_hbm, col_hbm, pr_hbm, pc_hbm, m_hbm,
                 e_hbm, denom_hbm,
                 prb, pcb, mb, rowc, colc, eb, dloc):
    cid = lax.axis_index("c")
    sid = lax.axis_index("s")
    w = cid * NS + sid

    pltpu.sync_copy(pr_hbm, prb)
    pltpu.sync_copy(pc_hbm, pcb)
    pltpu.sync_copy(m_hbm, mb)
    pltpu.sync_copy(row_hbm.at[pl.ds(w * CPW, CPW), :], rowc)
    pltpu.sync_copy(col_hbm.at[pl.ds(w * CPW, CPW), :], colc)

    def zero_d(j, _):
        dloc[pl.ds(j * 16, 16)] = jnp.zeros((16,), f32)
        return 0
    lax.fori_loop(0, NP // 16, zero_d, 0)

    mv = mb[...]
    lanes = lax.iota(i32, 16)
    base0 = w * CPW * C

    def chunk(i, _):
        def vec(j, _):
            sl = pl.ds(j * 16, 16)
            r16 = rowc[i, sl]
            c16 = colc[i, sl]
            u = plsc.load_gather(prb, [r16]) + plsc.load_gather(pcb, [c16])
            sim = jnp.where(u >= 0, u, 0.2 * u)
            ev = jnp.exp(sim - mv)
            gidx = base0 + i * C + j * 16 + lanes
            ev = jnp.where(gidx < NNZ, ev, 0.0)
            eb[i, sl] = ev
            plsc.addupdate_scatter(dloc, [r16], ev)
            return 0
        lax.fori_loop(0, C // 16, vec, 0)
        return 0
    lax.fori_loop(0, CPW, chunk, 0)

    pltpu.sync_copy(eb, e_hbm.at[pl.ds(w * CPW, CPW), :])
    pltpu.sync_copy(dloc, denom_hbm.at[w])


def _phase2(row2, col2, pr_p, pc_p, m16):
    kfn = pl.kernel(
        _phase2_body,
        out_type=[
            jax.ShapeDtypeStruct((NCHUNK, C), f32),
            jax.ShapeDtypeStruct((NW, NP), f32),
        ],
        mesh=_mesh(),
        compiler_params=_SC_PARAMS,
        scratch_types=[
            pltpu.VMEM((NP,), f32),
            pltpu.VMEM((NP,), f32),
            pltpu.VMEM((16,), f32),
            pltpu.VMEM((CPW, C), i32),
            pltpu.VMEM((CPW, C), i32),
            pltpu.VMEM((CPW, C), f32),
            pltpu.VMEM((NP,), f32),
        ],
    )
    return kfn(row2, col2, pr_p, pc_p, m16)


def _fold_denom_body(d_ref, out_ref):
    out_ref[...] = jnp.sum(d_ref[...], axis=0, keepdims=True)


def _fold_denom(denom32):
    return pl.pallas_call(
        _fold_denom_body,
        out_shape=jax.ShapeDtypeStruct((1, NP), f32),
    )(denom32)


def _phase3_body(row_hbm, col_hbm, e_hbm, denom_hbm, z1_hbm, z2_hbm,
                 att_hbm, de_hbm, t2_hbm,
                 d0b, rowc, colc, ec, attc, rowsb, deloc, zb,
                 t2sh, sem):
    cid = lax.axis_index("c")
    sid = lax.axis_index("s")
    w = cid * NS + sid

    pltpu.sync_copy(denom_hbm, d0b)
    pltpu.sync_copy(row_hbm.at[pl.ds(w * CPW, CPW), :], rowc)
    pltpu.sync_copy(col_hbm.at[pl.ds(w * CPW, CPW), :], colc)
    pltpu.sync_copy(e_hbm.at[pl.ds(w * CPW, CPW), :], ec)

    def zero_d(j, _):
        deloc[pl.ds(j * 16, 16)] = jnp.zeros((16,), f32)
        return 0
    lax.fori_loop(0, NP // 16, zero_d, 0)

    def zero_zb(j, _):
        q = j // (D2 // 16)
        r = j % (D2 // 16)
        zb[q, pl.ds(r * 16, 16)] = jnp.zeros((16,), f32)
        return 0
    lax.fori_loop(0, 64 * (D2 // 16), zero_zb, 0)

    def zero_acc():
        for k in range(RPT // 64):
            pltpu.sync_copy(zb, t2sh.at[pl.ds(sid * RPT + k * 64, 64), :])

    zero_acc()
    plsc.subcore_barrier()

    for h, zref in enumerate((z1_hbm, z2_hbm)):
        def chunk(i, _):
            if h == 0:
                def vec(j, _):
                    sl = pl.ds(j * 16, 16)
                    r16 = rowc[i, sl]
                    c16 = colc[i, sl]
                    dv = plsc.load_gather(d0b, [r16])
                    a16 = jnp.where(dv > 0, ec[i, sl] / dv, 0.0)
                    attc[i, sl] = a16
                    plsc.addupdate_scatter(deloc, [c16], a16)
                    return 0
                lax.fori_loop(0, C // 16, vec, 0)

            pltpu.async_copy(zref.at[rowc.at[i]], rowsb, sem).wait()

            def scale(b, _):
                i16 = jnp.broadcast_to(i.astype(i32), (16,))
                b16 = jnp.broadcast_to(b.astype(i32), (16,))
                spl = plsc.load_gather(attc, [i16, b16])
                for q in range(D2 // 16):
                    sl = pl.ds(q * 16, 16)
                    rowsb[b, sl] = rowsb[b, sl] * spl
                return 0
            lax.fori_loop(0, C, scale, 0)

            pltpu.sync_copy(rowsb, t2sh.at[colc.at[i]], add=True)
            return 0
        lax.fori_loop(0, CPW, chunk, 0)

        plsc.subcore_barrier()
        pltpu.sync_copy(t2sh.at[pl.ds(sid * RPT, RPT), :],
                        t2_hbm.at[h, cid, pl.ds(sid * RPT, RPT), :])
        if h == 0:
            zero_acc()
            plsc.subcore_barrier()

    pltpu.sync_copy(attc, att_hbm.at[pl.ds(w * CPW, CPW), :])
    pltpu.sync_copy(deloc, de_hbm.at[w])


def _phase3(row2, col2, e2, denom2, Z1, Z2):
    kfn = pl.kernel(
        _phase3_body,
        out_type=[
            jax.ShapeDtypeStruct((NCHUNK, C), f32),
            jax.ShapeDtypeStruct((NW, NP), f32),
            jax.ShapeDtypeStruct((2, 2, NP, D2), f32),
        ],
        mesh=_mesh(),
        compiler_params=_SC_PARAMS,
        scratch_types=[
            pltpu.VMEM((NP,), f32),
            pltpu.VMEM((CPW, C), i32),
            pltpu.VMEM((CPW, C), i32),
            pltpu.VMEM((CPW, C), f32),
            pltpu.VMEM((CPW, C), f32),
            pltpu.VMEM((C, D2), f32),
            pltpu.VMEM((NP,), f32),
            pltpu.VMEM((64, D2), f32),
            pltpu.VMEM_SHARED((NP, D2), f32),
            pltpu.SemaphoreType.DMA,
        ],
    )
    return kfn(row2, col2, e2, denom2, Z1, Z2)


def _fold_t3_body(t2_ref, de_ref, t3a_ref, t3b_ref):
    de = jnp.sum(de_ref[...], axis=0)[:E]
    deinv = jnp.where(de > 0, 1.0 / de, 0.0)[:, None]
    t3a_ref[...] = (t2_ref[0, 0] + t2_ref[0, 1])[:E] * deinv
    t3b_ref[...] = (t2_ref[1, 0] + t2_ref[1, 1])[:E] * deinv


def _fold_t3(t2p, de2):
    return pl.pallas_call(
        _fold_t3_body,
        out_shape=[
            jax.ShapeDtypeStruct((E, D2), f32),
            jax.ShapeDtypeStruct((E, D2), f32),
        ],
    )(t2p, de2)


def _phase5_body(row_hbm, col_hbm, att_hbm, t3a_hbm, t3b_hbm,
                 t4_hbm,
                 rowc, colc, attc, rowsb, zb, t4sh, sem):
    cid = lax.axis_index("c")
    sid = lax.axis_index("s")
    w = cid * NS + sid

    pltpu.sync_copy(row_hbm.at[pl.ds(w * CPW, CPW), :], rowc)
    pltpu.sync_copy(col_hbm.at[pl.ds(w * CPW, CPW), :], colc)
    pltpu.sync_copy(att_hbm.at[pl.ds(w * CPW, CPW), :], attc)

    def zero_zb(j, _):
        q = j // (D2 // 16)
        r = j % (D2 // 16)
        zb[q, pl.ds(r * 16, 16)] = jnp.zeros((16,), f32)
        return 0
    lax.fori_loop(0, 64 * (D2 // 16), zero_zb, 0)

    def zero_acc():
        for k in range(RPT // 64):
            pltpu.sync_copy(zb, t4sh.at[pl.ds(sid * RPT + k * 64, 64), :])

    zero_acc()
    plsc.subcore_barrier()

    for h, t3ref in enumerate((t3a_hbm, t3b_hbm)):
        def chunk(i, _):
            pltpu.async_copy(t3ref.at[colc.at[i]], rowsb, sem).wait()

            def scale(b, _):
                i16 = jnp.broadcast_to(i.astype(i32), (16,))
                b16 = jnp.broadcast_to(b.astype(i32), (16,))
                spl = plsc.load_gather(attc, [i16, b16])
                for q in range(D2 // 16):
                    sl = pl.ds(q * 16, 16)
                    rowsb[b, sl] = rowsb[b, sl] * spl
                return 0
            lax.fori_loop(0, C, scale, 0)

            pltpu.sync_copy(rowsb, t4sh.at[rowc.at[i]], add=True)
            return 0
        lax.fori_loop(0, CPW, chunk, 0)

        plsc.subcore_barrier()
        pltpu.sync_copy(t4sh.at[pl.ds(sid * RPT, RPT), :],
                        t4_hbm.at[h, cid, pl.ds(sid * RPT, RPT), :])
        if h == 0:
            zero_acc()
            plsc.subcore_barrier()


def _phase5(row2, col2, att2, t3a, t3b):
    kfn = pl.kernel(
        _phase5_body,
        out_type=jax.ShapeDtypeStruct((2, 2, NP, D2), f32),
        mesh=_mesh(),
        compiler_params=_SC_PARAMS,
        scratch_types=[
            pltpu.VMEM((CPW, C), i32),
            pltpu.VMEM((CPW, C), i32),
            pltpu.VMEM((CPW, C), f32),
            pltpu.VMEM((C, D2), f32),
            pltpu.VMEM((64, D2), f32),
            pltpu.VMEM_SHARED((NP, D2), f32),
            pltpu.SemaphoreType.DMA,
        ],
    )
    return kfn(row2, col2, att2, t3a, t3b)


def _fold_out_body(t4_ref, out_ref):
    a = (t4_ref[0, 0] + t4_ref[0, 1])[:N]
    b = (t4_ref[1, 0] + t4_ref[1, 1])[:N]
    out_ref[...] = jnp.concatenate([a, b], axis=1)


def _fold_out(t4p):
    return pl.pallas_call(
        _fold_out_body,
        out_shape=jax.ShapeDtypeStruct((N, D), f32),
    )(t4p)


def kernel(H, X, X_edges, P_w, P_b, a_w, a_b):
    row = H[0].astype(i32)
    col = H[1].astype(i32)
    row2 = jnp.pad(row, (0, NNZ_P - NNZ)).reshape(NCHUNK, C)
    col2 = jnp.pad(col, (0, NNZ_P - NNZ)).reshape(NCHUNK, C)

    Z, Ze, pr, pc, m = _dense(X, X_edges, P_w, P_b, a_w, a_b)
    pr_p = jnp.pad(pr[:, 0], (0, NP - N))
    pc_p = jnp.pad(pc[:, 0], (0, NP - E))
    m16 = jnp.broadcast_to(m.reshape(1), (16,))

    e2, denom32 = _phase2(row2, col2, pr_p, pc_p, m16)
    denom = _fold_denom(denom32).reshape(NP)
    att2, de32, t2p = _phase3(row2, col2, e2, denom,
                              Z[:, :D2], Z[:, D2:])
    t3a, t3b = _fold_t3(t2p, de32)
    t4p = _phase5(row2, col2, att2, t3a, t3b)
    return _fold_out(t4p)


# speedup vs baseline: 8.2746x; 1.2004x over previous
import jax
import jax.numpy as jnp
from jax import lax
from jax.experimental import pallas as pl
from jax.experimental.pallas import tpu as pltpu
from jax.experimental.pallas import tpu_sc as plsc

N = 10000
E = 10000
NNZ = 320000
D = 128
D2 = 32

NP = 10240
C = 128
NW = 32
CPW = 80
NNZ_P = NW * CPW * C
NCHUNK = NNZ_P // C
NS = 16
RPT = NP // NS

f32 = jnp.float32
i32 = jnp.int32

_SC_PARAMS = pltpu.CompilerParams(use_tc_tiling_on_sc=False,
                                  needs_layout_passes=False)


def _mesh():
    return plsc.VectorSubcoreMesh(core_axis_name="c", subcore_axis_name="s")


def _dense_body(x_ref, xe_ref, pwt_ref, pb_ref, w1_ref, w2_ref, ab_ref,
                z_ref, ze_ref, pr_ref, pc_ref, m_ref):
    z = jnp.dot(x_ref[...], pwt_ref[...], preferred_element_type=f32) + pb_ref[...]
    ze = jnp.dot(xe_ref[...], pwt_ref[...], preferred_element_type=f32) + pb_ref[...]
    z_ref[...] = z
    ze_ref[...] = ze
    pr = jnp.dot(z, w1_ref[...], preferred_element_type=f32)
    pc = jnp.dot(ze, w2_ref[...], preferred_element_type=f32) + ab_ref[...]
    pr_ref[...] = pr
    pc_ref[...] = pc
    m = jnp.max(pr) + jnp.max(pc)
    m = jnp.where(m >= 0, m, 0.2 * m)
    m_ref[...] = jnp.full((1, 1), m, f32)


def _dense(X, X_edges, P_w, P_b, a_w, a_b):
    return pl.pallas_call(
        _dense_body,
        out_shape=[
            jax.ShapeDtypeStruct((N, D), f32),
            jax.ShapeDtypeStruct((E, D), f32),
            jax.ShapeDtypeStruct((N, 1), f32),
            jax.ShapeDtypeStruct((E, 1), f32),
            jax.ShapeDtypeStruct((1, 1), f32),
        ],
    )(X, X_edges, P_w.T, P_b.reshape(1, D), a_w[0, :D].reshape(D, 1),
      a_w[0, D:].reshape(D, 1), a_b.reshape(1, 1))


def _phase2_body(row_hbm, col_hbm, pr_hbm, pc_hbm, m_hbm,
                 e_hbm, denom_hbm,
                 prb, pcb, mb, rowc, colc, eb, dloc):
    cid = lax.axis_index("c")
    sid = lax.axis_index("s")
    w = cid * NS + sid

    pltpu.sync_copy(pr_hbm, prb)
    pltpu.sync_copy(pc_hbm, pcb)
    pltpu.sync_copy(m_hbm, mb)
    pltpu.sync_copy(row_hbm.at[pl.ds(w * CPW, CPW), :], rowc)
    pltpu.sync_copy(col_hbm.at[pl.ds(w * CPW, CPW), :], colc)

    def zero_d(j, _):
        dloc[pl.ds(j * 16, 16)] = jnp.zeros((16,), f32)
        return 0
    lax.fori_loop(0, NP // 16, zero_d, 0)

    mv = mb[...]
    lanes = lax.iota(i32, 16)
    base0 = w * CPW * C

    def chunk(i, _):
        def vec(j, _):
            sl = pl.ds(j * 16, 16)
            r16 = rowc[i, sl]
            c16 = colc[i, sl]
            u = plsc.load_gather(prb, [r16]) + plsc.load_gather(pcb, [c16])
            sim = jnp.where(u >= 0, u, 0.2 * u)
            ev = jnp.exp(sim - mv)
            gidx = base0 + i * C + j * 16 + lanes
            ev = jnp.where(gidx < NNZ, ev, 0.0)
            eb[i, sl] = ev
            plsc.addupdate_scatter(dloc, [r16], ev)
            return 0
        lax.fori_loop(0, C // 16, vec, 0)
        return 0
    lax.fori_loop(0, CPW, chunk, 0)

    pltpu.sync_copy(eb, e_hbm.at[pl.ds(w * CPW, CPW), :])
    pltpu.sync_copy(dloc, denom_hbm.at[w])


def _phase2(row2, col2, pr_p, pc_p, m16):
    kfn = pl.kernel(
        _phase2_body,
        out_type=[
            jax.ShapeDtypeStruct((NCHUNK, C), f32),
            jax.ShapeDtypeStruct((NW, NP), f32),
        ],
        mesh=_mesh(),
        compiler_params=_SC_PARAMS,
        scratch_types=[
            pltpu.VMEM((NP,), f32),
            pltpu.VMEM((NP,), f32),
            pltpu.VMEM((16,), f32),
            pltpu.VMEM((CPW, C), i32),
            pltpu.VMEM((CPW, C), i32),
            pltpu.VMEM((CPW, C), f32),
            pltpu.VMEM((NP,), f32),
        ],
    )
    return kfn(row2, col2, pr_p, pc_p, m16)


def _fold_denom_body(d_ref, out_ref):
    out_ref[...] = jnp.sum(d_ref[...], axis=0, keepdims=True)


def _fold_denom(denom32):
    return pl.pallas_call(
        _fold_denom_body,
        out_shape=jax.ShapeDtypeStruct((1, NP), f32),
    )(denom32)


def _pipelined_pass(src_hbm, idx_gather, idx_scatter, attc, acc_sh,
                    gbuf0, gbuf1, sbuf0, sbuf1,
                    gsem0, gsem1, ssem0, ssem1):
    bufs = ((gbuf0, sbuf0, gsem0, ssem0), (gbuf1, sbuf1, gsem1, ssem1))

    pltpu.async_copy(src_hbm.at[idx_gather.at[0]], gbuf0, gsem0)
    pltpu.async_copy(src_hbm.at[idx_gather.at[1]], gbuf1, gsem1)

    def outer(g, _):
        for b, (gbuf, sbuf, gsem, ssem) in enumerate(bufs):
            c = g * 2 + b
            pltpu.make_async_copy(src_hbm.at[idx_gather.at[c]], gbuf,
                                  gsem).wait()

            @pl.when(g >= 1)
            def _():
                pltpu.make_async_copy(sbuf, acc_sh.at[idx_scatter.at[c]],
                                      ssem).wait()

            def scale(r, _):
                c16 = jnp.broadcast_to(c.astype(i32), (16,))
                r16 = jnp.broadcast_to(r.astype(i32), (16,))
                spl = plsc.load_gather(attc, [c16, r16])
                for q in range(D2 // 16):
                    sl = pl.ds(q * 16, 16)
                    sbuf[r, sl] = gbuf[r, sl] * spl
                return 0
            lax.fori_loop(0, C, scale, 0)

            @pl.when(c + 2 < CPW)
            def _():
                pltpu.async_copy(src_hbm.at[idx_gather.at[c + 2]], gbuf, gsem)

            pltpu.async_copy(sbuf, acc_sh.at[idx_scatter.at[c]], ssem,
                             add=True)
        return 0
    lax.fori_loop(0, CPW // 2, outer, 0)

    pltpu.make_async_copy(sbuf0, acc_sh.at[idx_scatter.at[CPW - 2]],
                          ssem0).wait()
    pltpu.make_async_copy(sbuf1, acc_sh.at[idx_scatter.at[CPW - 1]],
                          ssem1).wait()


def _phase3_body(row_hbm, col_hbm, e_hbm, denom_hbm,
                 z1_hbm, z2_hbm, z3_hbm, z4_hbm,
                 att_hbm, de_hbm, t2_hbm,
                 d0b, rowc, colc, ec, attc, deloc, zb,
                 gbuf0, gbuf1, sbuf0, sbuf1, t2sh,
                 gsem0, gsem1, ssem0, ssem1):
    cid = lax.axis_index("c")
    sid = lax.axis_index("s")
    w = cid * NS + sid

    pltpu.sync_copy(denom_hbm, d0b)
    pltpu.sync_copy(row_hbm.at[pl.ds(w * CPW, CPW), :], rowc)
    pltpu.sync_copy(col_hbm.at[pl.ds(w * CPW, CPW), :], colc)
    pltpu.sync_copy(e_hbm.at[pl.ds(w * CPW, CPW), :], ec)

    def zero_d(j, _):
        deloc[pl.ds(j * 16, 16)] = jnp.zeros((16,), f32)
        return 0
    lax.fori_loop(0, NP // 16, zero_d, 0)

    def zero_zb(j, _):
        q = j // (D2 // 16)
        r = j % (D2 // 16)
        zb[q, pl.ds(r * 16, 16)] = jnp.zeros((16,), f32)
        return 0
    lax.fori_loop(0, 64 * (D2 // 16), zero_zb, 0)

    def zero_acc():
        for k in range(RPT // 64):
            pltpu.sync_copy(zb, t2sh.at[pl.ds(sid * RPT + k * 64, 64), :])

    zero_acc()

    def att_chunk(i, _):
        def vec(j, _):
            sl = pl.ds(j * 16, 16)
            r16 = rowc[i, sl]
            c16 = colc[i, sl]
            dv = plsc.load_gather(d0b, [r16])
            a16 = jnp.where(dv > 0, ec[i, sl] / dv, 0.0)
            attc[i, sl] = a16
            plsc.addupdate_scatter(deloc, [c16], a16)
            return 0
        lax.fori_loop(0, C // 16, vec, 0)
        return 0
    lax.fori_loop(0, CPW, att_chunk, 0)

    pltpu.sync_copy(attc, att_hbm.at[pl.ds(w * CPW, CPW), :])
    pltpu.sync_copy(deloc, de_hbm.at[w])

    plsc.subcore_barrier()

    for h, zref in enumerate((z1_hbm, z2_hbm, z3_hbm, z4_hbm)):
        _pipelined_pass(zref, rowc, colc, attc, t2sh,
                        gbuf0, gbuf1, sbuf0, sbuf1,
                        gsem0, gsem1, ssem0, ssem1)

        plsc.subcore_barrier()
        pltpu.sync_copy(t2sh.at[pl.ds(sid * RPT, RPT), :],
                        t2_hbm.at[h, cid, pl.ds(sid * RPT, RPT), :])
        if h < 3:
            zero_acc()
            plsc.subcore_barrier()


def _phase3(row2, col2, e2, denom2, Z1, Z2, Z3, Z4):
    kfn = pl.kernel(
        _phase3_body,
        out_type=[
            jax.ShapeDtypeStruct((NCHUNK, C), f32),
            jax.ShapeDtypeStruct((NW, NP), f32),
            jax.ShapeDtypeStruct((4, 2, NP, D2), f32),
        ],
        mesh=_mesh(),
        compiler_params=_SC_PARAMS,
        scratch_types=[
            pltpu.VMEM((NP,), f32),
            pltpu.VMEM((CPW, C), i32),
            pltpu.VMEM((CPW, C), i32),
            pltpu.VMEM((CPW, C), f32),
            pltpu.VMEM((CPW, C), f32),
            pltpu.VMEM((NP,), f32),
            pltpu.VMEM((64, D2), f32),
            pltpu.VMEM((C, D2), f32),
            pltpu.VMEM((C, D2), f32),
            pltpu.VMEM((C, D2), f32),
            pltpu.VMEM((C, D2), f32),
            pltpu.VMEM_SHARED((NP, D2), f32),
            pltpu.SemaphoreType.DMA,
            pltpu.SemaphoreType.DMA,
            pltpu.SemaphoreType.DMA,
            pltpu.SemaphoreType.DMA,
        ],
    )
    return kfn(row2, col2, e2, denom2, Z1, Z2, Z3, Z4)


def _fold_t3_body(t2_ref, de_ref, sel_ref, t3a_ref, t3b_ref, t3c_ref,
                  t3d_ref):
    de = jnp.sum(de_ref[...], axis=0)
    deinv = jnp.where(de > 0, 1.0 / de, 0.0)
    dexp = jnp.dot(deinv, sel_ref[...], preferred_element_type=f32,
                   precision=lax.Precision.HIGHEST)
    for q, out in enumerate((t3a_ref, t3b_ref, t3c_ref, t3d_ref)):
        out[...] = ((t2_ref[q, 0] + t2_ref[q, 1]) * dexp)[:E // 4]


def _fold_t3(t2p, de2):
    sel = (jnp.arange(128)[None, :] // D2 == jnp.arange(4)[:, None]
           ).astype(f32)
    return pl.pallas_call(
        _fold_t3_body,
        out_shape=[jax.ShapeDtypeStruct((E // 4, D), f32)] * 4,
    )(t2p.reshape(4, 2, NP // 4, D), de2.reshape(NW, NP // 4, 4), sel)


def _phase5_body(row_hbm, col_hbm, att_hbm,
                 t3a_hbm, t3b_hbm, t3c_hbm, t3d_hbm,
                 t4_hbm,
                 rowc, colc, attc, zb,
                 gbuf0, gbuf1, sbuf0, sbuf1, t4sh,
                 gsem0, gsem1, ssem0, ssem1):
    cid = lax.axis_index("c")
    sid = lax.axis_index("s")
    w = cid * NS + sid

    pltpu.sync_copy(row_hbm.at[pl.ds(w * CPW, CPW), :], rowc)
    pltpu.sync_copy(col_hbm.at[pl.ds(w * CPW, CPW), :], colc)
    pltpu.sync_copy(att_hbm.at[pl.ds(w * CPW, CPW), :], attc)

    def zero_zb(j, _):
        q = j // (D2 // 16)
        r = j % (D2 // 16)
        zb[q, pl.ds(r * 16, 16)] = jnp.zeros((16,), f32)
        return 0
    lax.fori_loop(0, 64 * (D2 // 16), zero_zb, 0)

    def zero_acc():
        for k in range(RPT // 64):
            pltpu.sync_copy(zb, t4sh.at[pl.ds(sid * RPT + k * 64, 64), :])

    zero_acc()
    plsc.subcore_barrier()

    for h, t3ref in enumerate((t3a_hbm, t3b_hbm, t3c_hbm, t3d_hbm)):
        _pipelined_pass(t3ref, colc, rowc, attc, t4sh,
                        gbuf0, gbuf1, sbuf0, sbuf1,
                        gsem0, gsem1, ssem0, ssem1)

        plsc.subcore_barrier()
        pltpu.sync_copy(t4sh.at[pl.ds(sid * RPT, RPT), :],
                        t4_hbm.at[h, cid, pl.ds(sid * RPT, RPT), :])
        if h < 3:
            zero_acc()
            plsc.subcore_barrier()


def _phase5(row2, col2, att2, t3a, t3b, t3c, t3d):
    kfn = pl.kernel(
        _phase5_body,
        out_type=jax.ShapeDtypeStruct((4, 2, NP, D2), f32),
        mesh=_mesh(),
        compiler_params=_SC_PARAMS,
        scratch_types=[
            pltpu.VMEM((CPW, C), i32),
            pltpu.VMEM((CPW, C), i32),
            pltpu.VMEM((CPW, C), f32),
            pltpu.VMEM((64, D2), f32),
            pltpu.VMEM((C, D2), f32),
            pltpu.VMEM((C, D2), f32),
            pltpu.VMEM((C, D2), f32),
            pltpu.VMEM((C, D2), f32),
            pltpu.VMEM_SHARED((NP, D2), f32),
            pltpu.SemaphoreType.DMA,
            pltpu.SemaphoreType.DMA,
            pltpu.SemaphoreType.DMA,
            pltpu.SemaphoreType.DMA,
        ],
    )
    return kfn(row2, col2, att2, t3a, t3b, t3c, t3d)


def _fold_out_body(t4_ref, out_ref):
    parts = [(t4_ref[q, 0] + t4_ref[q, 1])[:N] for q in range(4)]
    out_ref[...] = jnp.concatenate(parts, axis=1)


def _fold_out(t4p):
    return pl.pallas_call(
        _fold_out_body,
        out_shape=jax.ShapeDtypeStruct((N, D), f32),
    )(t4p)


def kernel(H, X, X_edges, P_w, P_b, a_w, a_b):
    row = H[0].astype(i32)
    col = H[1].astype(i32)
    row2 = jnp.pad(row, (0, NNZ_P - NNZ)).reshape(NCHUNK, C)
    col2 = jnp.pad(col, (0, NNZ_P - NNZ)).reshape(NCHUNK, C)

    Z, Ze, pr, pc, m = _dense(X, X_edges, P_w, P_b, a_w, a_b)
    pr_p = jnp.pad(pr[:, 0], (0, NP - N))
    pc_p = jnp.pad(pc[:, 0], (0, NP - E))
    m16 = jnp.broadcast_to(m.reshape(1), (16,))

    e2, denom32 = _phase2(row2, col2, pr_p, pc_p, m16)
    denom = _fold_denom(denom32).reshape(NP)
    att2, de32, t2p = _phase3(row2, col2, e2, denom,
                              Z[:, 0 * D2:1 * D2], Z[:, 1 * D2:2 * D2],
                              Z[:, 2 * D2:3 * D2], Z[:, 3 * D2:4 * D2])
    t3w = _fold_t3(t2p, de32)
    t3a, t3b, t3c, t3d = (t.reshape(E, D2) for t in t3w)
    t4p = _phase5(row2, col2, att2, t3a, t3b, t3c, t3d)
    return _fold_out(t4p)


# speedup vs baseline: 9.2177x; 1.1140x over previous
import jax
import jax.numpy as jnp
from jax import lax
from jax.experimental import pallas as pl
from jax.experimental.pallas import tpu as pltpu
from jax.experimental.pallas import tpu_sc as plsc

N = 10000
E = 10000
NNZ = 320000
D = 128
D2 = 32

NP = 10240
C = 128
NW = 32
CPW = 80
NNZ_P = NW * CPW * C
NCHUNK = NNZ_P // C
NS = 16
RPT = NP // NS

f32 = jnp.float32
i32 = jnp.int32

_SC_PARAMS = pltpu.CompilerParams(use_tc_tiling_on_sc=False,
                                  needs_layout_passes=False)


def _mesh():
    return plsc.VectorSubcoreMesh(core_axis_name="c", subcore_axis_name="s")


def _dense_body(x_ref, xe_ref, pwt_ref, pb_ref, w1_ref, w2_ref, ab_ref,
                z_ref, ze_ref, pr_ref, pc_ref, m_ref):
    z = jnp.dot(x_ref[...], pwt_ref[...], preferred_element_type=f32) + pb_ref[...]
    ze = jnp.dot(xe_ref[...], pwt_ref[...], preferred_element_type=f32) + pb_ref[...]
    z_ref[...] = z
    ze_ref[...] = ze
    pr = jnp.dot(z, w1_ref[...], preferred_element_type=f32)
    pc = jnp.dot(ze, w2_ref[...], preferred_element_type=f32) + ab_ref[...]
    pr_ref[...] = pr
    pc_ref[...] = pc
    m = jnp.max(pr) + jnp.max(pc)
    m = jnp.where(m >= 0, m, 0.2 * m)
    m_ref[...] = jnp.full((1, 1), m, f32)


def _dense(X, X_edges, P_w, P_b, a_w, a_b):
    return pl.pallas_call(
        _dense_body,
        out_shape=[
            jax.ShapeDtypeStruct((N, D), f32),
            jax.ShapeDtypeStruct((E, D), f32),
            jax.ShapeDtypeStruct((N, 1), f32),
            jax.ShapeDtypeStruct((E, 1), f32),
            jax.ShapeDtypeStruct((1, 1), f32),
        ],
    )(X, X_edges, P_w.T, P_b.reshape(1, D), a_w[0, :D].reshape(D, 1),
      a_w[0, D:].reshape(D, 1), a_b.reshape(1, 1))


def _phase2_body(row_hbm, col_hbm, pr_hbm, pc_hbm, m_hbm,
                 e_hbm, denom_hbm,
                 prb, pcb, mb, rowc, colc, eb, dloc):
    cid = lax.axis_index("c")
    sid = lax.axis_index("s")
    w = cid * NS + sid

    pltpu.sync_copy(pr_hbm, prb)
    pltpu.sync_copy(pc_hbm, pcb)
    pltpu.sync_copy(m_hbm, mb)
    pltpu.sync_copy(row_hbm.at[pl.ds(w * CPW, CPW), :], rowc)
    pltpu.sync_copy(col_hbm.at[pl.ds(w * CPW, CPW), :], colc)

    def zero_d(j, _):
        dloc[pl.ds(j * 16, 16)] = jnp.zeros((16,), f32)
        return 0
    lax.fori_loop(0, NP // 16, zero_d, 0)

    mv = mb[...]
    lanes = lax.iota(i32, 16)
    base0 = w * CPW * C

    def chunk(i, _):
        def vec(j, _):
            sl = pl.ds(j * 16, 16)
            r16 = rowc[i, sl]
            c16 = colc[i, sl]
            u = plsc.load_gather(prb, [r16]) + plsc.load_gather(pcb, [c16])
            sim = jnp.where(u >= 0, u, 0.2 * u)
            ev = jnp.exp(sim - mv)
            gidx = base0 + i * C + j * 16 + lanes
            ev = jnp.where(gidx < NNZ, ev, 0.0)
            eb[i, sl] = ev
            plsc.addupdate_scatter(dloc, [r16], ev)
            return 0
        lax.fori_loop(0, C // 16, vec, 0)
        return 0
    lax.fori_loop(0, CPW, chunk, 0)

    pltpu.sync_copy(eb, e_hbm.at[pl.ds(w * CPW, CPW), :])
    pltpu.sync_copy(dloc, denom_hbm.at[w])


def _phase2(row2, col2, pr_p, pc_p, m16):
    kfn = pl.kernel(
        _phase2_body,
        out_type=[
            jax.ShapeDtypeStruct((NCHUNK, C), f32),
            jax.ShapeDtypeStruct((NW, NP), f32),
        ],
        mesh=_mesh(),
        compiler_params=_SC_PARAMS,
        scratch_types=[
            pltpu.VMEM((NP,), f32),
            pltpu.VMEM((NP,), f32),
            pltpu.VMEM((16,), f32),
            pltpu.VMEM((CPW, C), i32),
            pltpu.VMEM((CPW, C), i32),
            pltpu.VMEM((CPW, C), f32),
            pltpu.VMEM((NP,), f32),
        ],
    )
    return kfn(row2, col2, pr_p, pc_p, m16)


def _fold_denom_body(d_ref, out_ref):
    out_ref[...] = jnp.sum(d_ref[...], axis=0, keepdims=True)


def _fold_denom(denom32):
    return pl.pallas_call(
        _fold_denom_body,
        out_shape=jax.ShapeDtypeStruct((1, NP), f32),
    )(denom32)


def _pipelined_pass(src_hbm, idx_gather, idx_scatter, attc, acc_sh,
                    gbuf0, gbuf1, sbuf0, sbuf1,
                    gsem0, gsem1, ssem0, ssem1):
    bufs = ((gbuf0, sbuf0, gsem0, ssem0), (gbuf1, sbuf1, gsem1, ssem1))

    pltpu.async_copy(src_hbm.at[idx_gather.at[0]], gbuf0, gsem0)
    pltpu.async_copy(src_hbm.at[idx_gather.at[1]], gbuf1, gsem1)

    def outer(g, _):
        for b, (gbuf, sbuf, gsem, ssem) in enumerate(bufs):
            c = g * 2 + b
            pltpu.make_async_copy(src_hbm.at[idx_gather.at[c]], gbuf,
                                  gsem).wait()

            @pl.when(g >= 1)
            def _():
                pltpu.make_async_copy(sbuf, acc_sh.at[idx_scatter.at[c]],
                                      ssem).wait()

            def scale_block(r0, _):
                att16 = attc[c, pl.ds(r0 * 16, 16)]
                for j in range(16):
                    spl = jnp.broadcast_to(att16[j], (16,))
                    r = r0 * 16 + j
                    for q in range(D2 // 16):
                        sl = pl.ds(q * 16, 16)
                        sbuf[r, sl] = gbuf[r, sl] * spl
                return 0
            lax.fori_loop(0, C // 16, scale_block, 0)

            @pl.when(c + 2 < CPW)
            def _():
                pltpu.async_copy(src_hbm.at[idx_gather.at[c + 2]], gbuf, gsem)

            pltpu.async_copy(sbuf, acc_sh.at[idx_scatter.at[c]], ssem,
                             add=True)
        return 0
    lax.fori_loop(0, CPW // 2, outer, 0)

    pltpu.make_async_copy(sbuf0, acc_sh.at[idx_scatter.at[CPW - 2]],
                          ssem0).wait()
    pltpu.make_async_copy(sbuf1, acc_sh.at[idx_scatter.at[CPW - 1]],
                          ssem1).wait()


def _phase3_body(row_hbm, col_hbm, e_hbm, denom_hbm,
                 z1_hbm, z2_hbm, z3_hbm, z4_hbm,
                 att_hbm, de_hbm, t2_hbm,
                 d0b, rowc, colc, ec, attc, deloc, zb,
                 gbuf0, gbuf1, sbuf0, sbuf1, t2sh,
                 gsem0, gsem1, ssem0, ssem1):
    cid = lax.axis_index("c")
    sid = lax.axis_index("s")
    w = cid * NS + sid

    pltpu.sync_copy(denom_hbm, d0b)
    pltpu.sync_copy(row_hbm.at[pl.ds(w * CPW, CPW), :], rowc)
    pltpu.sync_copy(col_hbm.at[pl.ds(w * CPW, CPW), :], colc)
    pltpu.sync_copy(e_hbm.at[pl.ds(w * CPW, CPW), :], ec)

    def zero_d(j, _):
        deloc[pl.ds(j * 16, 16)] = jnp.zeros((16,), f32)
        return 0
    lax.fori_loop(0, NP // 16, zero_d, 0)

    def zero_zb(j, _):
        q = j // (D2 // 16)
        r = j % (D2 // 16)
        zb[q, pl.ds(r * 16, 16)] = jnp.zeros((16,), f32)
        return 0
    lax.fori_loop(0, 64 * (D2 // 16), zero_zb, 0)

    def zero_acc():
        for k in range(RPT // 64):
            pltpu.sync_copy(zb, t2sh.at[pl.ds(sid * RPT + k * 64, 64), :])

    zero_acc()

    def att_chunk(i, _):
        def vec(j, _):
            sl = pl.ds(j * 16, 16)
            r16 = rowc[i, sl]
            c16 = colc[i, sl]
            dv = plsc.load_gather(d0b, [r16])
            a16 = jnp.where(dv > 0, ec[i, sl] / dv, 0.0)
            attc[i, sl] = a16
            plsc.addupdate_scatter(deloc, [c16], a16)
            return 0
        lax.fori_loop(0, C // 16, vec, 0)
        return 0
    lax.fori_loop(0, CPW, att_chunk, 0)

    pltpu.sync_copy(attc, att_hbm.at[pl.ds(w * CPW, CPW), :])
    pltpu.sync_copy(deloc, de_hbm.at[w])

    plsc.subcore_barrier()

    for h, zref in enumerate((z1_hbm, z2_hbm, z3_hbm, z4_hbm)):
        _pipelined_pass(zref, rowc, colc, attc, t2sh,
                        gbuf0, gbuf1, sbuf0, sbuf1,
                        gsem0, gsem1, ssem0, ssem1)

        plsc.subcore_barrier()
        pltpu.sync_copy(t2sh.at[pl.ds(sid * RPT, RPT), :],
                        t2_hbm.at[h, cid, pl.ds(sid * RPT, RPT), :])
        if h < 3:
            zero_acc()
            plsc.subcore_barrier()


def _phase3(row2, col2, e2, denom2, Z1, Z2, Z3, Z4):
    kfn = pl.kernel(
        _phase3_body,
        out_type=[
            jax.ShapeDtypeStruct((NCHUNK, C), f32),
            jax.ShapeDtypeStruct((NW, NP), f32),
            jax.ShapeDtypeStruct((4, 2, NP, D2), f32),
        ],
        mesh=_mesh(),
        compiler_params=_SC_PARAMS,
        scratch_types=[
            pltpu.VMEM((NP,), f32),
            pltpu.VMEM((CPW, C), i32),
            pltpu.VMEM((CPW, C), i32),
            pltpu.VMEM((CPW, C), f32),
            pltpu.VMEM((CPW, C), f32),
            pltpu.VMEM((NP,), f32),
            pltpu.VMEM((64, D2), f32),
            pltpu.VMEM((C, D2), f32),
            pltpu.VMEM((C, D2), f32),
            pltpu.VMEM((C, D2), f32),
            pltpu.VMEM((C, D2), f32),
            pltpu.VMEM_SHARED((NP, D2), f32),
            pltpu.SemaphoreType.DMA,
            pltpu.SemaphoreType.DMA,
            pltpu.SemaphoreType.DMA,
            pltpu.SemaphoreType.DMA,
        ],
    )
    return kfn(row2, col2, e2, denom2, Z1, Z2, Z3, Z4)


def _fold_t3_body(t2_ref, de_ref, sel_ref, t3a_ref, t3b_ref, t3c_ref,
                  t3d_ref):
    de = jnp.sum(de_ref[...], axis=0)
    deinv = jnp.where(de > 0, 1.0 / de, 0.0)
    dexp = jnp.dot(deinv, sel_ref[...], preferred_element_type=f32,
                   precision=lax.Precision.HIGHEST)
    for q, out in enumerate((t3a_ref, t3b_ref, t3c_ref, t3d_ref)):
        out[...] = ((t2_ref[q, 0] + t2_ref[q, 1]) * dexp)[:E // 4]


def _fold_t3(t2p, de2):
    sel = (jnp.arange(128)[None, :] // D2 == jnp.arange(4)[:, None]
           ).astype(f32)
    return pl.pallas_call(
        _fold_t3_body,
        out_shape=[jax.ShapeDtypeStruct((E // 4, D), f32)] * 4,
    )(t2p.reshape(4, 2, NP // 4, D), de2.reshape(NW, NP // 4, 4), sel)


def _phase5_body(row_hbm, col_hbm, att_hbm,
                 t3a_hbm, t3b_hbm, t3c_hbm, t3d_hbm,
                 t4_hbm,
                 rowc, colc, attc, zb,
                 gbuf0, gbuf1, sbuf0, sbuf1, t4sh,
                 gsem0, gsem1, ssem0, ssem1):
    cid = lax.axis_index("c")
    sid = lax.axis_index("s")
    w = cid * NS + sid

    pltpu.sync_copy(row_hbm.at[pl.ds(w * CPW, CPW), :], rowc)
    pltpu.sync_copy(col_hbm.at[pl.ds(w * CPW, CPW), :], colc)
    pltpu.sync_copy(att_hbm.at[pl.ds(w * CPW, CPW), :], attc)

    def zero_zb(j, _):
        q = j // (D2 // 16)
        r = j % (D2 // 16)
        zb[q, pl.ds(r * 16, 16)] = jnp.zeros((16,), f32)
        return 0
    lax.fori_loop(0, 64 * (D2 // 16), zero_zb, 0)

    def zero_acc():
        for k in range(RPT // 64):
            pltpu.sync_copy(zb, t4sh.at[pl.ds(sid * RPT + k * 64, 64), :])

    zero_acc()
    plsc.subcore_barrier()

    for h, t3ref in enumerate((t3a_hbm, t3b_hbm, t3c_hbm, t3d_hbm)):
        _pipelined_pass(t3ref, colc, rowc, attc, t4sh,
                        gbuf0, gbuf1, sbuf0, sbuf1,
                        gsem0, gsem1, ssem0, ssem1)

        plsc.subcore_barrier()
        pltpu.sync_copy(t4sh.at[pl.ds(sid * RPT, RPT), :],
                        t4_hbm.at[h, cid, pl.ds(sid * RPT, RPT), :])
        if h < 3:
            zero_acc()
            plsc.subcore_barrier()


def _phase5(row2, col2, att2, t3a, t3b, t3c, t3d):
    kfn = pl.kernel(
        _phase5_body,
        out_type=jax.ShapeDtypeStruct((4, 2, NP, D2), f32),
        mesh=_mesh(),
        compiler_params=_SC_PARAMS,
        scratch_types=[
            pltpu.VMEM((CPW, C), i32),
            pltpu.VMEM((CPW, C), i32),
            pltpu.VMEM((CPW, C), f32),
            pltpu.VMEM((64, D2), f32),
            pltpu.VMEM((C, D2), f32),
            pltpu.VMEM((C, D2), f32),
            pltpu.VMEM((C, D2), f32),
            pltpu.VMEM((C, D2), f32),
            pltpu.VMEM_SHARED((NP, D2), f32),
            pltpu.SemaphoreType.DMA,
            pltpu.SemaphoreType.DMA,
            pltpu.SemaphoreType.DMA,
            pltpu.SemaphoreType.DMA,
        ],
    )
    return kfn(row2, col2, att2, t3a, t3b, t3c, t3d)


def _fold_out_body(t4_ref, out_ref):
    parts = [(t4_ref[q, 0] + t4_ref[q, 1])[:N] for q in range(4)]
    out_ref[...] = jnp.concatenate(parts, axis=1)


def _fold_out(t4p):
    return pl.pallas_call(
        _fold_out_body,
        out_shape=jax.ShapeDtypeStruct((N, D), f32),
    )(t4p)


def kernel(H, X, X_edges, P_w, P_b, a_w, a_b):
    row = H[0].astype(i32)
    col = H[1].astype(i32)
    row2 = jnp.pad(row, (0, NNZ_P - NNZ)).reshape(NCHUNK, C)
    col2 = jnp.pad(col, (0, NNZ_P - NNZ)).reshape(NCHUNK, C)

    Z, Ze, pr, pc, m = _dense(X, X_edges, P_w, P_b, a_w, a_b)
    pr_p = jnp.pad(pr[:, 0], (0, NP - N))
    pc_p = jnp.pad(pc[:, 0], (0, NP - E))
    m16 = jnp.broadcast_to(m.reshape(1), (16,))

    e2, denom32 = _phase2(row2, col2, pr_p, pc_p, m16)
    denom = _fold_denom(denom32).reshape(NP)
    att2, de32, t2p = _phase3(row2, col2, e2, denom,
                              Z[:, 0 * D2:1 * D2], Z[:, 1 * D2:2 * D2],
                              Z[:, 2 * D2:3 * D2], Z[:, 3 * D2:4 * D2])
    t3w = _fold_t3(t2p, de32)
    t3a, t3b, t3c, t3d = (t.reshape(E, D2) for t in t3w)
    t4p = _phase5(row2, col2, att2, t3a, t3b, t3c, t3d)
    return _fold_out(t4p)


# speedup vs baseline: 9.4751x; 1.0279x over previous
import jax
import jax.numpy as jnp
from jax import lax
from jax.experimental import pallas as pl
from jax.experimental.pallas import tpu as pltpu
from jax.experimental.pallas import tpu_sc as plsc

N = 10000
E = 10000
NNZ = 320000
D = 128
D2 = 32

NP = 10240
C = 128
NW = 32
CPW = 80
NNZ_P = NW * CPW * C
NCHUNK = NNZ_P // C
NS = 16
RPT = NP // NS

f32 = jnp.float32
i32 = jnp.int32

_SC_PARAMS = pltpu.CompilerParams(use_tc_tiling_on_sc=False,
                                  needs_layout_passes=False)


def _mesh():
    return plsc.VectorSubcoreMesh(core_axis_name="c", subcore_axis_name="s")


def _dense_body(x_ref, xe_ref, pwt_ref, pb_ref, w1_ref, w2_ref, ab_ref,
                z_ref, ze_ref, pr_ref, pc_ref, m_ref):
    z = jnp.dot(x_ref[...], pwt_ref[...], preferred_element_type=f32) + pb_ref[...]
    ze = jnp.dot(xe_ref[...], pwt_ref[...], preferred_element_type=f32) + pb_ref[...]
    z_ref[...] = z
    ze_ref[...] = ze
    pr = jnp.dot(z, w1_ref[...], preferred_element_type=f32)
    pc = jnp.dot(ze, w2_ref[...], preferred_element_type=f32) + ab_ref[...]
    pr_ref[...] = pr
    pc_ref[...] = pc
    m = jnp.max(pr) + jnp.max(pc)
    m = jnp.where(m >= 0, m, 0.2 * m)
    m_ref[...] = jnp.full((1, 1), m, f32)


def _dense(X, X_edges, P_w, P_b, a_w, a_b):
    return pl.pallas_call(
        _dense_body,
        out_shape=[
            jax.ShapeDtypeStruct((N, D), f32),
            jax.ShapeDtypeStruct((E, D), f32),
            jax.ShapeDtypeStruct((N, 1), f32),
            jax.ShapeDtypeStruct((E, 1), f32),
            jax.ShapeDtypeStruct((1, 1), f32),
        ],
    )(X, X_edges, P_w.T, P_b.reshape(1, D), a_w[0, :D].reshape(D, 1),
      a_w[0, D:].reshape(D, 1), a_b.reshape(1, 1))


def _phase2_body(row_hbm, col_hbm, pr_hbm, pc_hbm, m_hbm,
                 e_hbm, denom_hbm,
                 prb, pcb, mb, rowc, colc, eb, dloc):
    cid = lax.axis_index("c")
    sid = lax.axis_index("s")
    w = cid * NS + sid

    pltpu.sync_copy(pr_hbm, prb)
    pltpu.sync_copy(pc_hbm, pcb)
    pltpu.sync_copy(m_hbm, mb)
    pltpu.sync_copy(row_hbm.at[pl.ds(w * CPW, CPW), :], rowc)
    pltpu.sync_copy(col_hbm.at[pl.ds(w * CPW, CPW), :], colc)

    def zero_d(j, _):
        dloc[pl.ds(j * 16, 16)] = jnp.zeros((16,), f32)
        return 0
    lax.fori_loop(0, NP // 16, zero_d, 0)

    mv = mb[...]
    lanes = lax.iota(i32, 16)
    base0 = w * CPW * C

    def chunk(i, _):
        def vec(j, _):
            sl = pl.ds(j * 16, 16)
            r16 = rowc[i, sl]
            c16 = colc[i, sl]
            u = plsc.load_gather(prb, [r16]) + plsc.load_gather(pcb, [c16])
            sim = jnp.where(u >= 0, u, 0.2 * u)
            ev = jnp.exp(sim - mv)
            gidx = base0 + i * C + j * 16 + lanes
            ev = jnp.where(gidx < NNZ, ev, 0.0)
            eb[i, sl] = ev
            plsc.addupdate_scatter(dloc, [r16], ev)
            return 0
        lax.fori_loop(0, C // 16, vec, 0)
        return 0
    lax.fori_loop(0, CPW, chunk, 0)

    pltpu.sync_copy(eb, e_hbm.at[pl.ds(w * CPW, CPW), :])
    pltpu.sync_copy(dloc, denom_hbm.at[w])


def _phase2(row2, col2, pr_p, pc_p, m16):
    kfn = pl.kernel(
        _phase2_body,
        out_type=[
            jax.ShapeDtypeStruct((NCHUNK, C), f32),
            jax.ShapeDtypeStruct((NW, NP), f32),
        ],
        mesh=_mesh(),
        compiler_params=_SC_PARAMS,
        scratch_types=[
            pltpu.VMEM((NP,), f32),
            pltpu.VMEM((NP,), f32),
            pltpu.VMEM((16,), f32),
            pltpu.VMEM((CPW, C), i32),
            pltpu.VMEM((CPW, C), i32),
            pltpu.VMEM((CPW, C), f32),
            pltpu.VMEM((NP,), f32),
        ],
    )
    return kfn(row2, col2, pr_p, pc_p, m16)


def _fold_denom_body(d_ref, out_ref):
    out_ref[...] = jnp.sum(d_ref[...], axis=0, keepdims=True)


def _fold_denom(denom32):
    return pl.pallas_call(
        _fold_denom_body,
        out_shape=jax.ShapeDtypeStruct((1, NP), f32),
    )(denom32)


NB = 4


def _pipelined_pass(src_hbm, idx_gather, idx_scatter, attc, acc_sh,
                    gbufs, sbufs, gsems, ssems):
    for b in range(NB):
        pltpu.async_copy(src_hbm.at[idx_gather.at[b]], gbufs[b], gsems[b])

    def outer(g, _):
        for b in range(NB):
            gbuf, sbuf, gsem, ssem = gbufs[b], sbufs[b], gsems[b], ssems[b]
            c = g * NB + b
            pltpu.make_async_copy(src_hbm.at[idx_gather.at[c]], gbuf,
                                  gsem).wait()

            @pl.when(g >= 1)
            def _():
                pltpu.make_async_copy(sbuf, acc_sh.at[idx_scatter.at[c]],
                                      ssem).wait()

            def scale_block(r0, _):
                att16 = attc[c, pl.ds(r0 * 16, 16)]
                for j in range(16):
                    spl = jnp.broadcast_to(att16[j], (16,))
                    r = r0 * 16 + j
                    for q in range(D2 // 16):
                        sl = pl.ds(q * 16, 16)
                        sbuf[r, sl] = gbuf[r, sl] * spl
                return 0
            lax.fori_loop(0, C // 16, scale_block, 0)

            @pl.when(c + NB < CPW)
            def _():
                pltpu.async_copy(src_hbm.at[idx_gather.at[c + NB]], gbuf,
                                 gsem)

            pltpu.async_copy(sbuf, acc_sh.at[idx_scatter.at[c]], ssem,
                             add=True)
        return 0
    lax.fori_loop(0, CPW // NB, outer, 0)

    for b in range(NB):
        pltpu.make_async_copy(sbufs[b], acc_sh.at[idx_scatter.at[CPW - NB + b]],
                              ssems[b]).wait()


def _phase3_body(row_hbm, col_hbm, e_hbm, denom_hbm,
                 z1_hbm, z2_hbm, z3_hbm, z4_hbm,
                 att_hbm, de_hbm, t2_hbm,
                 d0b, rowc, colc, ec, attc, deloc, zb,
                 gbuf0, gbuf1, gbuf2, gbuf3, sbuf0, sbuf1, sbuf2, sbuf3, t2sh,
                 gsem0, gsem1, gsem2, gsem3, ssem0, ssem1, ssem2, ssem3):
    cid = lax.axis_index("c")
    sid = lax.axis_index("s")
    w = cid * NS + sid

    pltpu.sync_copy(denom_hbm, d0b)
    pltpu.sync_copy(row_hbm.at[pl.ds(w * CPW, CPW), :], rowc)
    pltpu.sync_copy(col_hbm.at[pl.ds(w * CPW, CPW), :], colc)
    pltpu.sync_copy(e_hbm.at[pl.ds(w * CPW, CPW), :], ec)

    def zero_d(j, _):
        deloc[pl.ds(j * 16, 16)] = jnp.zeros((16,), f32)
        return 0
    lax.fori_loop(0, NP // 16, zero_d, 0)

    def zero_zb(j, _):
        q = j // (D2 // 16)
        r = j % (D2 // 16)
        zb[q, pl.ds(r * 16, 16)] = jnp.zeros((16,), f32)
        return 0
    lax.fori_loop(0, 64 * (D2 // 16), zero_zb, 0)

    def zero_acc():
        for k in range(RPT // 64):
            pltpu.sync_copy(zb, t2sh.at[pl.ds(sid * RPT + k * 64, 64), :])

    zero_acc()

    def att_chunk(i, _):
        def vec(j, _):
            sl = pl.ds(j * 16, 16)
            r16 = rowc[i, sl]
            c16 = colc[i, sl]
            dv = plsc.load_gather(d0b, [r16])
            a16 = jnp.where(dv > 0, ec[i, sl] / dv, 0.0)
            attc[i, sl] = a16
            plsc.addupdate_scatter(deloc, [c16], a16)
            return 0
        lax.fori_loop(0, C // 16, vec, 0)
        return 0
    lax.fori_loop(0, CPW, att_chunk, 0)

    pltpu.sync_copy(attc, att_hbm.at[pl.ds(w * CPW, CPW), :])
    pltpu.sync_copy(deloc, de_hbm.at[w])

    plsc.subcore_barrier()

    for h, zref in enumerate((z1_hbm, z2_hbm, z3_hbm, z4_hbm)):
        _pipelined_pass(zref, rowc, colc, attc, t2sh,
                        (gbuf0, gbuf1, gbuf2, gbuf3),
                        (sbuf0, sbuf1, sbuf2, sbuf3),
                        (gsem0, gsem1, gsem2, gsem3),
                        (ssem0, ssem1, ssem2, ssem3))

        plsc.subcore_barrier()
        pltpu.sync_copy(t2sh.at[pl.ds(sid * RPT, RPT), :],
                        t2_hbm.at[h, cid, pl.ds(sid * RPT, RPT), :])
        if h < 3:
            zero_acc()
            plsc.subcore_barrier()


def _phase3(row2, col2, e2, denom2, Z1, Z2, Z3, Z4):
    kfn = pl.kernel(
        _phase3_body,
        out_type=[
            jax.ShapeDtypeStruct((NCHUNK, C), f32),
            jax.ShapeDtypeStruct((NW, NP), f32),
            jax.ShapeDtypeStruct((4, 2, NP, D2), f32),
        ],
        mesh=_mesh(),
        compiler_params=_SC_PARAMS,
        scratch_types=[
            pltpu.VMEM((NP,), f32),
            pltpu.VMEM((CPW, C), i32),
            pltpu.VMEM((CPW, C), i32),
            pltpu.VMEM((CPW, C), f32),
            pltpu.VMEM((CPW, C), f32),
            pltpu.VMEM((NP,), f32),
            pltpu.VMEM((64, D2), f32),
            pltpu.VMEM((C, D2), f32),
            pltpu.VMEM((C, D2), f32),
            pltpu.VMEM((C, D2), f32),
            pltpu.VMEM((C, D2), f32),
            pltpu.VMEM((C, D2), f32),
            pltpu.VMEM((C, D2), f32),
            pltpu.VMEM((C, D2), f32),
            pltpu.VMEM((C, D2), f32),
            pltpu.VMEM_SHARED((NP, D2), f32),
            pltpu.SemaphoreType.DMA,
            pltpu.SemaphoreType.DMA,
            pltpu.SemaphoreType.DMA,
            pltpu.SemaphoreType.DMA,
            pltpu.SemaphoreType.DMA,
            pltpu.SemaphoreType.DMA,
            pltpu.SemaphoreType.DMA,
            pltpu.SemaphoreType.DMA,
        ],
    )
    return kfn(row2, col2, e2, denom2, Z1, Z2, Z3, Z4)


def _fold_t3_body(t2_ref, de_ref, sel_ref, t3a_ref, t3b_ref, t3c_ref,
                  t3d_ref):
    de = jnp.sum(de_ref[...], axis=0)
    deinv = jnp.where(de > 0, 1.0 / de, 0.0)
    dexp = jnp.dot(deinv, sel_ref[...], preferred_element_type=f32,
                   precision=lax.Precision.HIGHEST)
    for q, out in enumerate((t3a_ref, t3b_ref, t3c_ref, t3d_ref)):
        out[...] = ((t2_ref[q, 0] + t2_ref[q, 1]) * dexp)[:E // 4]


def _fold_t3(t2p, de2):
    sel = (jnp.arange(128)[None, :] // D2 == jnp.arange(4)[:, None]
           ).astype(f32)
    return pl.pallas_call(
        _fold_t3_body,
        out_shape=[jax.ShapeDtypeStruct((E // 4, D), f32)] * 4,
    )(t2p.reshape(4, 2, NP // 4, D), de2.reshape(NW, NP // 4, 4), sel)


def _phase5_body(row_hbm, col_hbm, att_hbm,
                 t3a_hbm, t3b_hbm, t3c_hbm, t3d_hbm,
                 t4_hbm,
                 rowc, colc, attc, zb,
                 gbuf0, gbuf1, gbuf2, gbuf3, sbuf0, sbuf1, sbuf2, sbuf3, t4sh,
                 gsem0, gsem1, gsem2, gsem3, ssem0, ssem1, ssem2, ssem3):
    cid = lax.axis_index("c")
    sid = lax.axis_index("s")
    w = cid * NS + sid

    pltpu.sync_copy(row_hbm.at[pl.ds(w * CPW, CPW), :], rowc)
    pltpu.sync_copy(col_hbm.at[pl.ds(w * CPW, CPW), :], colc)
    pltpu.sync_copy(att_hbm.at[pl.ds(w * CPW, CPW), :], attc)

    def zero_zb(j, _):
        q = j // (D2 // 16)
        r = j % (D2 // 16)
        zb[q, pl.ds(r * 16, 16)] = jnp.zeros((16,), f32)
        return 0
    lax.fori_loop(0, 64 * (D2 // 16), zero_zb, 0)

    def zero_acc():
        for k in range(RPT // 64):
            pltpu.sync_copy(zb, t4sh.at[pl.ds(sid * RPT + k * 64, 64), :])

    zero_acc()
    plsc.subcore_barrier()

    for h, t3ref in enumerate((t3a_hbm, t3b_hbm, t3c_hbm, t3d_hbm)):
        _pipelined_pass(t3ref, colc, rowc, attc, t4sh,
                        (gbuf0, gbuf1, gbuf2, gbuf3),
                        (sbuf0, sbuf1, sbuf2, sbuf3),
                        (gsem0, gsem1, gsem2, gsem3),
                        (ssem0, ssem1, ssem2, ssem3))

        plsc.subcore_barrier()
        pltpu.sync_copy(t4sh.at[pl.ds(sid * RPT, RPT), :],
                        t4_hbm.at[h, cid, pl.ds(sid * RPT, RPT), :])
        if h < 3:
            zero_acc()
            plsc.subcore_barrier()


def _phase5(row2, col2, att2, t3a, t3b, t3c, t3d):
    kfn = pl.kernel(
        _phase5_body,
        out_type=jax.ShapeDtypeStruct((4, 2, NP, D2), f32),
        mesh=_mesh(),
        compiler_params=_SC_PARAMS,
        scratch_types=[
            pltpu.VMEM((CPW, C), i32),
            pltpu.VMEM((CPW, C), i32),
            pltpu.VMEM((CPW, C), f32),
            pltpu.VMEM((64, D2), f32),
            pltpu.VMEM((C, D2), f32),
            pltpu.VMEM((C, D2), f32),
            pltpu.VMEM((C, D2), f32),
            pltpu.VMEM((C, D2), f32),
            pltpu.VMEM((C, D2), f32),
            pltpu.VMEM((C, D2), f32),
            pltpu.VMEM((C, D2), f32),
            pltpu.VMEM((C, D2), f32),
            pltpu.VMEM_SHARED((NP, D2), f32),
            pltpu.SemaphoreType.DMA,
            pltpu.SemaphoreType.DMA,
            pltpu.SemaphoreType.DMA,
            pltpu.SemaphoreType.DMA,
            pltpu.SemaphoreType.DMA,
            pltpu.SemaphoreType.DMA,
            pltpu.SemaphoreType.DMA,
            pltpu.SemaphoreType.DMA,
        ],
    )
    return kfn(row2, col2, att2, t3a, t3b, t3c, t3d)


def _fold_out_body(t4_ref, out_ref):
    parts = [(t4_ref[q, 0] + t4_ref[q, 1])[:N] for q in range(4)]
    out_ref[...] = jnp.concatenate(parts, axis=1)


def _fold_out(t4p):
    return pl.pallas_call(
        _fold_out_body,
        out_shape=jax.ShapeDtypeStruct((N, D), f32),
    )(t4p)


def kernel(H, X, X_edges, P_w, P_b, a_w, a_b):
    row = H[0].astype(i32)
    col = H[1].astype(i32)
    row2 = jnp.pad(row, (0, NNZ_P - NNZ)).reshape(NCHUNK, C)
    col2 = jnp.pad(col, (0, NNZ_P - NNZ)).reshape(NCHUNK, C)

    Z, Ze, pr, pc, m = _dense(X, X_edges, P_w, P_b, a_w, a_b)
    pr_p = jnp.pad(pr[:, 0], (0, NP - N))
    pc_p = jnp.pad(pc[:, 0], (0, NP - E))
    m16 = jnp.broadcast_to(m.reshape(1), (16,))

    e2, denom32 = _phase2(row2, col2, pr_p, pc_p, m16)
    denom = _fold_denom(denom32).reshape(NP)
    att2, de32, t2p = _phase3(row2, col2, e2, denom,
                              Z[:, 0 * D2:1 * D2], Z[:, 1 * D2:2 * D2],
                              Z[:, 2 * D2:3 * D2], Z[:, 3 * D2:4 * D2])
    t3w = _fold_t3(t2p, de32)
    t3a, t3b, t3c, t3d = (t.reshape(E, D2) for t in t3w)
    t4p = _phase5(row2, col2, att2, t3a, t3b, t3c, t3d)
    return _fold_out(t4p)
